# baseline (speedup 1.0000x reference)
import functools

import jax
import jax.numpy as jnp
from jax import lax
from jax.experimental import pallas as pl
from jax.experimental.pallas import tpu as pltpu

_ALPHA = 2.0 / 1001.0


def _tile_body(c, sv_ref, w_ref, lab_ref, cent_ref, init_ref, out_ref):
    x = jnp.abs(sv_ref[...] * w_ref[...])
    b = x.shape[0]
    labs = lab_ref[0, :]
    onehot = (labs[None, :] == lax.broadcasted_iota(jnp.int32, (c, b), 0))
    onehot_f = onehot.astype(jnp.float32)
    sums = jnp.dot(onehot_f.astype(jnp.bfloat16), x.astype(jnp.bfloat16),
                   preferred_element_type=jnp.float32)
    counts = jnp.sum(onehot_f, axis=1, keepdims=True)
    mean = sums / jnp.maximum(counts, 1.0)
    cent = cent_ref[...]
    lerped = cent + (mean - cent) * _ALPHA
    upd = jnp.where(init_ref[...] > 0.0, lerped, mean)
    out_ref[...] = jnp.where(counts > 0.0, upd, cent)


def kernel(sparse_vector, W_eff, labels, centroids, initialized):
    b, v = sparse_vector.shape
    c = centroids.shape[0]
    tv = 2048
    num_tiles = pl.cdiv(v, tv)

    lab2d = labels.reshape(1, b)
    init_f = initialized.astype(jnp.float32).reshape(c, 1)

    grid_spec = pl.GridSpec(
        grid=(num_tiles,),
        in_specs=[
            pl.BlockSpec((b, tv), lambda i: (0, i)),
            pl.BlockSpec((b, tv), lambda i: (0, i)),
            pl.BlockSpec((1, b), lambda i: (0, 0)),
            pl.BlockSpec((c, tv), lambda i: (0, i)),
            pl.BlockSpec((c, 1), lambda i: (0, 0)),
        ],
        out_specs=pl.BlockSpec((c, tv), lambda i: (0, i)),
    )
    out = pl.pallas_call(
        functools.partial(_tile_body, c),
        grid_spec=grid_spec,
        out_shape=jax.ShapeDtypeStruct((c, v), jnp.float32),
        compiler_params=pltpu.CompilerParams(
            dimension_semantics=("parallel",),
            vmem_limit_bytes=100 * 1024 * 1024),
    )(sparse_vector, W_eff, lab2d, centroids, init_f)
    return out
